# baseline (speedup 1.0000x reference)
import functools

import jax
import jax.numpy as jnp
from jax import lax
from jax.experimental import pallas as pl
from jax.experimental.pallas import tpu as pltpu
from jax.experimental.pallas import tpu_sc as plsc

D = 128
G = 256
EPS = 1e-5

NC = 2
NS = 16
NW = NC * NS
CHUNK = 128


def _ln_tanh(z, g, b):
    mu = jnp.mean(z, axis=-1, keepdims=True)
    var = jnp.mean((z - mu) ** 2, axis=-1, keepdims=True)
    return jnp.tanh((z - mu) * jax.lax.rsqrt(var + EPS) * g + b)


def _full(shape):
    return pl.BlockSpec(shape, lambda i: tuple(0 for _ in shape))


def _node0(x, Wa, ba, WmA0, WmB0):
    N = x.shape[0]
    R = 2000
    def body(x_r, Wa_r, ba_r, WmA_r, WmB_r, h_r, a_r, b_r):
        h = jnp.dot(x_r[...], Wa_r[...], preferred_element_type=jnp.float32) + ba_r[...]
        h_r[...] = h
        a_r[...] = jnp.dot(h, WmA_r[...], preferred_element_type=jnp.float32)
        b_r[...] = jnp.dot(h, WmB_r[...], preferred_element_type=jnp.float32)
    out = jax.ShapeDtypeStruct((N, D), jnp.float32)
    return pl.pallas_call(
        body,
        grid=(N // R,),
        in_specs=[pl.BlockSpec((R, D), lambda i: (i, 0)), _full((D, D)),
                  _full((1, D)), _full((D, D)), _full((D, D))],
        out_specs=[pl.BlockSpec((R, D), lambda i: (i, 0))] * 3,
        out_shape=[out, out, out],
    )(x, Wa, ba, WmA0, WmB0)


def _edgefold(ea, Wb, bb, WmC0, bm0, WmC1, bm1):
    E, DE = ea.shape
    R = 4000
    def body(ea_r, Wb_r, bb_r, C0_r, b0_r, C1_r, b1_r, e0_r, e1_r):
        e = jnp.dot(ea_r[...], Wb_r[...], preferred_element_type=jnp.float32) + bb_r[...]
        e0_r[...] = jnp.dot(e, C0_r[...], preferred_element_type=jnp.float32) + b0_r[...]
        e1_r[...] = jnp.dot(e, C1_r[...], preferred_element_type=jnp.float32) + b1_r[...]
    out = jax.ShapeDtypeStruct((E, D), jnp.float32)
    return pl.pallas_call(
        body,
        grid=(E // R,),
        in_specs=[pl.BlockSpec((R, DE), lambda i: (i, 0)), _full((DE, DE)),
                  _full((1, DE)), _full((DE, D)), _full((1, D)),
                  _full((DE, D)), _full((1, D))],
        out_specs=[pl.BlockSpec((R, D), lambda i: (i, 0))] * 2,
        out_shape=[out, out],
    )(ea, Wb, bb, WmC0, bm0, WmC1, bm1)


def _update(P, h, WuA, WuB, bu, gu, beu, WmA, WmB):
    N = h.shape[0]
    R = 2000
    def body(P_r, h_r, WuA_r, WuB_r, bu_r, gu_r, beu_r, WmA_r, WmB_r,
             h1_r, a_r, b_r):
        aggr = P_r[0] + P_r[1]
        z = (jnp.dot(aggr, WuA_r[...], preferred_element_type=jnp.float32)
             + jnp.dot(h_r[...], WuB_r[...], preferred_element_type=jnp.float32)
             + bu_r[...])
        h1 = _ln_tanh(z, gu_r[...], beu_r[...])
        h1_r[...] = h1
        a_r[...] = jnp.dot(h1, WmA_r[...], preferred_element_type=jnp.float32)
        b_r[...] = jnp.dot(h1, WmB_r[...], preferred_element_type=jnp.float32)
    out = jax.ShapeDtypeStruct((N, D), jnp.float32)
    return pl.pallas_call(
        body,
        grid=(N // R,),
        in_specs=[pl.BlockSpec((2, R, D), lambda i: (0, i, 0)),
                  pl.BlockSpec((R, D), lambda i: (i, 0)),
                  _full((D, D)), _full((D, D)), _full((1, D)),
                  _full((1, D)), _full((1, D)), _full((D, D)), _full((D, D))],
        out_specs=[pl.BlockSpec((R, D), lambda i: (i, 0))] * 3,
        out_shape=[out, out, out],
    )(P, h, WuA, WuB, bu, gu, beu, WmA, WmB)


def _final(P, h, WuA, WuB, bu, gu, beu, W1, b1, W2, b2, batch2d):
    N = h.shape[0]
    R = 2000
    nblk = N // R
    def body(P_r, h_r, WuA_r, WuB_r, bu_r, gu_r, beu_r,
             W1_r, b1_r, W2_r, b2_r, bi_r, out_r, sums, cnts):
        i = pl.program_id(0)
        aggr = P_r[0] + P_r[1]
        z = (jnp.dot(aggr, WuA_r[...], preferred_element_type=jnp.float32)
             + jnp.dot(h_r[...], WuB_r[...], preferred_element_type=jnp.float32)
             + bu_r[...])
        h2 = _ln_tanh(z, gu_r[...], beu_r[...])
        hid = jax.nn.relu(jnp.dot(h2, W1_r[...], preferred_element_type=jnp.float32)
                          + b1_r[...])
        r = jnp.dot(hid, W2_r[...], preferred_element_type=jnp.float32) + b2_r[...]
        gids = jax.lax.broadcasted_iota(jnp.int32, (R, G), 1)
        mask = (bi_r[...] == gids).astype(jnp.float32)
        blk_sum = jax.lax.dot_general(
            r, mask, (((0,), (0,)), ((), ())), preferred_element_type=jnp.float32)
        blk_cnt = jnp.sum(mask, axis=0, keepdims=True)

        @pl.when(i == 0)
        def _():
            sums[...] = jnp.zeros_like(sums)
            cnts[...] = jnp.zeros_like(cnts)
        sums[...] += blk_sum
        cnts[...] += blk_cnt

        @pl.when(i == nblk - 1)
        def _():
            out_r[...] = sums[...] / jnp.maximum(cnts[...], 1.0)
    return pl.pallas_call(
        body,
        grid=(nblk,),
        in_specs=[pl.BlockSpec((2, R, D), lambda i: (0, i, 0)),
                  pl.BlockSpec((R, D), lambda i: (i, 0)),
                  _full((D, D)), _full((D, D)), _full((1, D)),
                  _full((1, D)), _full((1, D)),
                  _full((D, D)), _full((1, D)), _full((D, 1)), _full((1, 1)),
                  pl.BlockSpec((R, 1), lambda i: (i, 0))],
        out_specs=_full((1, G)),
        out_shape=jax.ShapeDtypeStruct((1, G), jnp.float32),
        scratch_shapes=[pltpu.VMEM((1, G), jnp.float32),
                        pltpu.VMEM((1, G), jnp.float32)],
    )(P, h, WuA, WuB, bu, gu, beu, W1, b1, W2, b2, batch2d)


def _sc_msgpass(T, Ee, cidx, sidx, gm, bem):
    N2, _ = T.shape
    N = N2 // 2
    n_chunks, CW = cidx.shape
    CH = CW // 2
    PER = n_chunks // NW
    n_valid_chunks = Ee.shape[0] // CH
    GRP = 16
    NGRP = PER // GRP
    row_step = 624
    row_span = 640
    mesh = plsc.VectorSubcoreMesh(core_axis_name="c", subcore_axis_name="s",
                                  num_cores=NC, num_subcores=NS)

    @functools.partial(
        pl.kernel, out_type=jax.ShapeDtypeStruct((NC, N, D), jnp.float32),
        mesh=mesh,
        scratch_types=[
            pltpu.VMEM_SHARED((N, D), jnp.float32),
            pltpu.VMEM((2, GRP, CW), jnp.int32),
            pltpu.VMEM((2, GRP, CH), jnp.int32),
            pltpu.VMEM((2, 2 * CH, D), jnp.float32),
            pltpu.VMEM((2, CH, D), jnp.float32),
            pltpu.VMEM((D,), jnp.float32),
            pltpu.VMEM((D,), jnp.float32),
            pltpu.SemaphoreType.DMA,
            pltpu.SemaphoreType.DMA,
            pltpu.SemaphoreType.DMA,
            pltpu.SemaphoreType.DMA,
        ])
    def k(T_h, Ee_h, cidx_h, sidx_h, gm_h, bem_h, P_h,
          aggr, idxg, sidxg, bufab, bufe, gvm, bvm,
          sg0, sg1, se0, se1):
        c = lax.axis_index("c")
        s = lax.axis_index("s")
        wid = s * NC + c
        chunk0 = wid * PER
        gsems = (sg0, sg1)
        esems = (se0, se1)
        nvalid = jnp.minimum(jnp.maximum(n_valid_chunks - chunk0, 0), PER)

        pltpu.sync_copy(gm_h, gvm)
        pltpu.sync_copy(bem_h, bvm)

        def zero_row(i, carry):
            for j in range(D // 16):
                bufab[0, i, pl.ds(j * 16, 16)] = jnp.zeros((16,), jnp.float32)
            return carry

        lax.fori_loop(0, 2 * CH, zero_row, 0)
        for kk in range(row_span // (2 * CH)):
            pltpu.sync_copy(bufab.at[0],
                            aggr.at[pl.ds(s * row_step + kk * 2 * CH, 2 * CH)])
        plsc.subcore_barrier()

        def stage(g):
            pg = lax.rem(g, 2)
            base = chunk0 + g * GRP
            pltpu.sync_copy(cidx_h.at[pl.ds(base, GRP)], idxg.at[pg])
            pltpu.sync_copy(sidx_h.at[pl.ds(base, GRP)], sidxg.at[pg])

        def gidx(kk):
            return idxg.at[lax.rem(kk // GRP, 2), lax.rem(kk, GRP)]

        def fire(kk, b):
            pltpu.async_copy(T_h.at[gidx(kk)], bufab.at[b], gsems[b])
            pltpu.async_copy(Ee_h.at[pl.ds((chunk0 + kk) * CH, CH)],
                             bufe.at[b], esems[b])

        def drain(kk, b):
            pltpu.make_async_copy(T_h.at[gidx(kk)], bufab.at[b], gsems[b]).wait()
            pltpu.make_async_copy(Ee_h.at[pl.ds((chunk0 + kk) * CH, CH)],
                                  bufe.at[b], esems[b]).wait()

        gs = [gvm[pl.ds(j * 16, 16)] for j in range(D // 16)]
        bs = [bvm[pl.ds(j * 16, 16)] for j in range(D // 16)]
        inv_d = 1.0 / D
        lane = lax.iota(jnp.int32, 16)
        bfly = [lane ^ sh for sh in (8, 4, 2, 1)]

        UNROLL = 4

        def compute(b):
            def row_group(i, carry):
                for rr in range(UNROLL):
                    _one_row(b, i * UNROLL + rr)
                return carry

            def _one_row(b, r):
                z = [bufab[b, r, pl.ds(j * 16, 16)]
                     + bufab[b, CH + r, pl.ds(j * 16, 16)]
                     + bufe[b, r, pl.ds(j * 16, 16)]
                     for j in range(D // 16)]
                s1 = z[0]
                s2 = z[0] * z[0]
                for j in range(1, D // 16):
                    s1 = s1 + z[j]
                    s2 = s2 + z[j] * z[j]
                for idx in bfly:
                    s1 = s1 + s1[idx]
                    s2 = s2 + s2[idx]
                mu = s1 * inv_d
                var = s2 * inv_d - mu * mu
                v = var + EPS
                yi = (jnp.int32(0x5F3759DF)
                      - (lax.bitcast_convert_type(v, jnp.int32) >> 1))
                y = lax.bitcast_convert_type(yi, jnp.float32)
                for _ in range(3):
                    y = y * (1.5 - 0.5 * v * y * y)
                for j in range(D // 16):
                    u = (z[j] - mu) * y * gs[j] + bs[j]
                    t = jnp.exp(-2.0 * jnp.abs(u))
                    m = (1.0 - t) / (1.0 + t)
                    m = jnp.where(u < 0.0, -m, m)
                    bufab[b, r, pl.ds(j * 16, 16)] = m

            lax.fori_loop(0, CH // UNROLL, row_group, 0)

        def scatter(kk, b):
            sref = sidxg.at[lax.rem(kk // GRP, 2), lax.rem(kk, GRP)]
            pltpu.sync_copy(bufab.at[b, pl.ds(0, CH)], aggr.at[sref], add=True)

        @pl.when(nvalid > 0)
        def _():
            stage(0)
            fire(0, 0)

        @pl.when(nvalid > 1)
        def _():
            fire(1, 1)

        def outer(i, carry):
            for b in range(2):
                kk = i * 2 + b

                @pl.when(kk < nvalid)
                def _():
                    g = kk // GRP

                    @pl.when((lax.rem(kk, GRP) == 0) & (g + 1 < NGRP))
                    def _():
                        stage(g + 1)

                    drain(kk, b)
                    compute(b)
                    scatter(kk, b)

                    @pl.when(kk + 2 < nvalid)
                    def _():
                        fire(kk + 2, b)
            return carry

        lax.fori_loop(0, PER // 2, outer, 0)
        plsc.subcore_barrier()
        pltpu.sync_copy(aggr.at[pl.ds(s * row_step, row_span)],
                        P_h.at[c, pl.ds(s * row_step, row_span)])

    return k(T, Ee, cidx, sidx, gm, bem)


def kernel(x, edge_index, edge_attr, batch_idx, Wa, ba, Wb, bb,
           Wm0, bm0, gm0, betam0, Wu0, bu0, gu0, betau0,
           Wm1, bm1, gm1, betam1, Wu1, bu1, gu1, betau1,
           W1, b1, W2, b2):
    N = x.shape[0]
    E = edge_index.shape[1]

    def row(v):
        return v.reshape(1, -1)

    CH = 40
    n_chunks = E // CH
    n_pad = -(-(-(-n_chunks // NW)) // 16) * 16 * NW
    src3 = edge_index[0].reshape(n_chunks, CH)
    dst3 = edge_index[1].reshape(n_chunks, CH)
    cidx = jnp.pad(jnp.concatenate([src3, dst3 + N], axis=1),
                   ((0, n_pad - n_chunks), (0, 0)))
    sidx = jnp.pad(dst3, ((0, n_pad - n_chunks), (0, 0)))

    h, A, B = _node0(x, Wa, row(ba), Wm0[:D], Wm0[D:2 * D])
    Ee0, Ee1 = _edgefold(edge_attr, Wb, row(bb),
                         Wm0[2 * D:], row(bm0), Wm1[2 * D:], row(bm1))

    P = _sc_msgpass(jnp.concatenate([A, B], axis=0), Ee0, cidx, sidx,
                    gm0, betam0)
    h, A, B = _update(P, h, Wu0[:D], Wu0[D:], row(bu0), row(gu0), row(betau0),
                      Wm1[:D], Wm1[D:2 * D])

    P = _sc_msgpass(jnp.concatenate([A, B], axis=0), Ee1, cidx, sidx,
                    gm1, betam1)

    return _final(P, h, Wu1[:D], Wu1[D:], row(bu1), row(gu1), row(betau1),
                  W1, row(b1), W2.reshape(D, 1), b2.reshape(1, 1),
                  batch_idx.reshape(N, 1))


# speedup vs baseline: 1.5661x; 1.5661x over previous
import functools

import jax
import jax.numpy as jnp
from jax import lax
from jax.experimental import pallas as pl
from jax.experimental.pallas import tpu as pltpu
from jax.experimental.pallas import tpu_sc as plsc

D = 128
G = 256
EPS = 1e-5

NC = 2
NS = 16
NW = NC * NS
CHUNK = 128


def _ln_tanh(z, g, b):
    mu = jnp.mean(z, axis=-1, keepdims=True)
    var = jnp.mean((z - mu) ** 2, axis=-1, keepdims=True)
    return jnp.tanh((z - mu) * jax.lax.rsqrt(var + EPS) * g + b)


def _full(shape):
    return pl.BlockSpec(shape, lambda i: tuple(0 for _ in shape))


def _node0(x, Wa, ba, WmA0, WmB0):
    N = x.shape[0]
    R = 2000
    def body(x_r, Wa_r, ba_r, WmA_r, WmB_r, h_r, a_r, b_r):
        h = jnp.dot(x_r[...], Wa_r[...], preferred_element_type=jnp.float32) + ba_r[...]
        h_r[...] = h
        a_r[...] = jnp.dot(h, WmA_r[...], preferred_element_type=jnp.float32)
        b_r[...] = jnp.dot(h, WmB_r[...], preferred_element_type=jnp.float32)
    out = jax.ShapeDtypeStruct((N, D), jnp.float32)
    return pl.pallas_call(
        body,
        grid=(N // R,),
        in_specs=[pl.BlockSpec((R, D), lambda i: (i, 0)), _full((D, D)),
                  _full((1, D)), _full((D, D)), _full((D, D))],
        out_specs=[pl.BlockSpec((R, D), lambda i: (i, 0))] * 3,
        out_shape=[out, out, out],
    )(x, Wa, ba, WmA0, WmB0)


def _edgefold(ea, Wb, bb, WmC0, bm0, WmC1, bm1):
    E, DE = ea.shape
    R = 4000
    def body(ea_r, Wb_r, bb_r, C0_r, b0_r, C1_r, b1_r, e0_r, e1_r):
        e = jnp.dot(ea_r[...], Wb_r[...], preferred_element_type=jnp.float32) + bb_r[...]
        e0_r[...] = jnp.dot(e, C0_r[...], preferred_element_type=jnp.float32) + b0_r[...]
        e1_r[...] = jnp.dot(e, C1_r[...], preferred_element_type=jnp.float32) + b1_r[...]
    out = jax.ShapeDtypeStruct((E, D), jnp.float32)
    return pl.pallas_call(
        body,
        grid=(E // R,),
        in_specs=[pl.BlockSpec((R, DE), lambda i: (i, 0)), _full((DE, DE)),
                  _full((1, DE)), _full((DE, D)), _full((1, D)),
                  _full((DE, D)), _full((1, D))],
        out_specs=[pl.BlockSpec((R, D), lambda i: (i, 0))] * 2,
        out_shape=[out, out],
    )(ea, Wb, bb, WmC0, bm0, WmC1, bm1)


def _msg(ga, gb, ee, gm, bem):
    E = ga.shape[0]
    R = 4000
    def body(ga_r, gb_r, ee_r, g_r, b_r, m_r):
        z = ga_r[...] + gb_r[...] + ee_r[...]
        m_r[...] = _ln_tanh(z, g_r[...], b_r[...])
    return pl.pallas_call(
        body,
        grid=(E // R,),
        in_specs=[pl.BlockSpec((R, D), lambda i: (i, 0))] * 3 + [_full((1, D))] * 2,
        out_specs=pl.BlockSpec((R, D), lambda i: (i, 0)),
        out_shape=jax.ShapeDtypeStruct((E, D), jnp.float32),
    )(ga, gb, ee, gm, bem)


def _update(P, h, WuA, WuB, bu, gu, beu, WmA, WmB):
    N = h.shape[0]
    R = 2000
    def body(P_r, h_r, WuA_r, WuB_r, bu_r, gu_r, beu_r, WmA_r, WmB_r,
             h1_r, a_r, b_r):
        aggr = P_r[0] + P_r[1]
        z = (jnp.dot(aggr, WuA_r[...], preferred_element_type=jnp.float32)
             + jnp.dot(h_r[...], WuB_r[...], preferred_element_type=jnp.float32)
             + bu_r[...])
        h1 = _ln_tanh(z, gu_r[...], beu_r[...])
        h1_r[...] = h1
        a_r[...] = jnp.dot(h1, WmA_r[...], preferred_element_type=jnp.float32)
        b_r[...] = jnp.dot(h1, WmB_r[...], preferred_element_type=jnp.float32)
    out = jax.ShapeDtypeStruct((N, D), jnp.float32)
    return pl.pallas_call(
        body,
        grid=(N // R,),
        in_specs=[pl.BlockSpec((2, R, D), lambda i: (0, i, 0)),
                  pl.BlockSpec((R, D), lambda i: (i, 0)),
                  _full((D, D)), _full((D, D)), _full((1, D)),
                  _full((1, D)), _full((1, D)), _full((D, D)), _full((D, D))],
        out_specs=[pl.BlockSpec((R, D), lambda i: (i, 0))] * 3,
        out_shape=[out, out, out],
    )(P, h, WuA, WuB, bu, gu, beu, WmA, WmB)


def _final(P, h, WuA, WuB, bu, gu, beu, W1, b1, W2, b2, batch2d):
    N = h.shape[0]
    R = 2000
    nblk = N // R
    def body(P_r, h_r, WuA_r, WuB_r, bu_r, gu_r, beu_r,
             W1_r, b1_r, W2_r, b2_r, bi_r, out_r, sums, cnts):
        i = pl.program_id(0)
        aggr = P_r[0] + P_r[1]
        z = (jnp.dot(aggr, WuA_r[...], preferred_element_type=jnp.float32)
             + jnp.dot(h_r[...], WuB_r[...], preferred_element_type=jnp.float32)
             + bu_r[...])
        h2 = _ln_tanh(z, gu_r[...], beu_r[...])
        hid = jax.nn.relu(jnp.dot(h2, W1_r[...], preferred_element_type=jnp.float32)
                          + b1_r[...])
        r = jnp.dot(hid, W2_r[...], preferred_element_type=jnp.float32) + b2_r[...]
        gids = jax.lax.broadcasted_iota(jnp.int32, (R, G), 1)
        mask = (bi_r[...] == gids).astype(jnp.float32)
        blk_sum = jax.lax.dot_general(
            r, mask, (((0,), (0,)), ((), ())), preferred_element_type=jnp.float32)
        blk_cnt = jnp.sum(mask, axis=0, keepdims=True)

        @pl.when(i == 0)
        def _():
            sums[...] = jnp.zeros_like(sums)
            cnts[...] = jnp.zeros_like(cnts)
        sums[...] += blk_sum
        cnts[...] += blk_cnt

        @pl.when(i == nblk - 1)
        def _():
            out_r[...] = sums[...] / jnp.maximum(cnts[...], 1.0)
    return pl.pallas_call(
        body,
        grid=(nblk,),
        in_specs=[pl.BlockSpec((2, R, D), lambda i: (0, i, 0)),
                  pl.BlockSpec((R, D), lambda i: (i, 0)),
                  _full((D, D)), _full((D, D)), _full((1, D)),
                  _full((1, D)), _full((1, D)),
                  _full((D, D)), _full((1, D)), _full((D, 1)), _full((1, 1)),
                  pl.BlockSpec((R, 1), lambda i: (i, 0))],
        out_specs=_full((1, G)),
        out_shape=jax.ShapeDtypeStruct((1, G), jnp.float32),
        scratch_shapes=[pltpu.VMEM((1, G), jnp.float32),
                        pltpu.VMEM((1, G), jnp.float32)],
    )(P, h, WuA, WuB, bu, gu, beu, W1, b1, W2, b2, batch2d)


def _sc_gather(A, B, src2d, dst2d, E):
    n_pad, C = src2d.shape
    PER = n_pad // NW
    n_chunks = E // C
    NB = 3
    mesh = plsc.VectorSubcoreMesh(core_axis_name="c", subcore_axis_name="s",
                                  num_cores=NC, num_subcores=NS)
    out = jax.ShapeDtypeStruct((E, D), jnp.float32)

    @functools.partial(
        pl.kernel, out_type=(out, out), mesh=mesh,
        scratch_types=[
            pltpu.VMEM((PER, C), jnp.int32),
            pltpu.VMEM((PER, C), jnp.int32),
            pltpu.VMEM((NB, C, D), jnp.float32),
            pltpu.VMEM((NB, C, D), jnp.float32),
            [pltpu.SemaphoreType.DMA] * NB,
            [pltpu.SemaphoreType.DMA] * NB,
            [pltpu.SemaphoreType.DMA] * NB,
            [pltpu.SemaphoreType.DMA] * NB,
        ])
    def k(A_h, B_h, src_h, dst_h, GA_h, GB_h,
          idxs, idxd, bufa, bufb, sga, sgb, swa, swb):
        c = lax.axis_index("c")
        s = lax.axis_index("s")
        wid = s * NC + c
        chunk0 = wid * PER
        nvalid = jnp.minimum(jnp.maximum(n_chunks - chunk0, 0), PER)

        pltpu.sync_copy(src_h.at[pl.ds(chunk0, PER)], idxs)
        pltpu.sync_copy(dst_h.at[pl.ds(chunk0, PER)], idxd)

        def fire_gather(kk, b):
            pltpu.async_copy(A_h.at[idxs.at[kk]], bufa.at[b], sga[b])
            pltpu.async_copy(B_h.at[idxd.at[kk]], bufb.at[b], sgb[b])

        def drain_gather(kk, b):
            pltpu.make_async_copy(A_h.at[idxs.at[kk]], bufa.at[b], sga[b]).wait()
            pltpu.make_async_copy(B_h.at[idxd.at[kk]], bufb.at[b], sgb[b]).wait()

        def fire_wb(kk, b):
            base = (chunk0 + kk) * C
            pltpu.async_copy(bufa.at[b], GA_h.at[pl.ds(base, C)], swa[b])
            pltpu.async_copy(bufb.at[b], GB_h.at[pl.ds(base, C)], swb[b])

        def drain_wb(kk, b):
            base = (chunk0 + kk) * C
            pltpu.make_async_copy(bufa.at[b], GA_h.at[pl.ds(base, C)], swa[b]).wait()
            pltpu.make_async_copy(bufb.at[b], GB_h.at[pl.ds(base, C)], swb[b]).wait()

        @pl.when(nvalid > 0)
        def _():
            fire_gather(0, 0)

        @pl.when(nvalid > 1)
        def _():
            fire_gather(1, 1)

        def outer(i, carry):
            for bb in range(NB):
                kk = i * NB + bb
                bn = (bb + 2) % NB

                @pl.when(kk < nvalid)
                def _():
                    drain_gather(kk, bb)
                    fire_wb(kk, bb)

                    @pl.when((kk + 2 < nvalid) & (kk > 0))
                    def _():
                        drain_wb(kk - 1, bn)
                        fire_gather(kk + 2, bn)

                    @pl.when((kk + 2 < nvalid) & (kk == 0))
                    def _():
                        fire_gather(kk + 2, bn)
            return carry

        lax.fori_loop(0, -(-(PER + 2) // NB), outer, 0)

        for bb in range(NB):
            ct = nvalid - 1 - lax.rem(nvalid - 1 - bb + 2 * NB, NB)

            @pl.when((ct >= 0) & (ct >= nvalid - NB))
            def _():
                drain_wb(ct, bb)

    return k(A, B, src2d, dst2d)


def _sc_scatter(M, dst, N):
    E = dst.shape[0]
    C = CHUNK
    n_chunks = E // C
    PER = -(-n_chunks // NW)
    row_step = 624
    row_span = 640
    mesh = plsc.VectorSubcoreMesh(core_axis_name="c", subcore_axis_name="s",
                                  num_cores=NC, num_subcores=NS)

    @functools.partial(
        pl.kernel, out_type=jax.ShapeDtypeStruct((NC, N, D), jnp.float32),
        mesh=mesh,
        scratch_types=[
            pltpu.VMEM_SHARED((N, D), jnp.float32),
            pltpu.VMEM((2, C, D), jnp.float32),
            pltpu.VMEM((C,), jnp.int32),
            pltpu.VMEM((C,), jnp.int32),
            [pltpu.SemaphoreType.DMA] * 2,
            [pltpu.SemaphoreType.DMA] * 2,
        ])
    def k(M_h, dst_h, P_h, aggr, bufm, idx0, idx1, sm, si):
        c = lax.axis_index("c")
        s = lax.axis_index("s")
        wid = s * NC + c
        chunk0 = wid * PER
        nvalid = jnp.minimum(jnp.maximum(n_chunks - chunk0, 0), PER)
        idxb = (idx0, idx1)

        def zero_row(i, carry):
            for j in range(D // 16):
                bufm[0, i, pl.ds(j * 16, 16)] = jnp.zeros((16,), jnp.float32)
            return carry

        lax.fori_loop(0, C, zero_row, 0)
        for kk in range(row_span // C):
            pltpu.sync_copy(bufm.at[0],
                            aggr.at[pl.ds(s * row_step + kk * C, C)])
        plsc.subcore_barrier()

        def fire(kk, b):
            base = (chunk0 + kk) * C
            pltpu.async_copy(M_h.at[pl.ds(base, C)], bufm.at[b], sm[b])
            pltpu.async_copy(dst_h.at[pl.ds(base, C)], idxb[b], si[b])

        def drain(kk, b):
            base = (chunk0 + kk) * C
            pltpu.make_async_copy(M_h.at[pl.ds(base, C)], bufm.at[b], sm[b]).wait()
            pltpu.make_async_copy(dst_h.at[pl.ds(base, C)], idxb[b], si[b]).wait()

        @pl.when(nvalid > 0)
        def _():
            fire(0, 0)

        @pl.when(nvalid > 1)
        def _():
            fire(1, 1)

        def outer(i, carry):
            for b in range(2):
                kk = i * 2 + b

                @pl.when(kk < nvalid)
                def _():
                    drain(kk, b)
                    pltpu.sync_copy(bufm.at[b], aggr.at[idxb[b]], add=True)

                    @pl.when(kk + 2 < nvalid)
                    def _():
                        fire(kk + 2, b)
            return carry

        lax.fori_loop(0, -(-PER // 2), outer, 0)
        plsc.subcore_barrier()
        pltpu.sync_copy(aggr.at[pl.ds(s * row_step, row_span)],
                        P_h.at[c, pl.ds(s * row_step, row_span)])

    return k(M, dst)


def kernel(x, edge_index, edge_attr, batch_idx, Wa, ba, Wb, bb,
           Wm0, bm0, gm0, betam0, Wu0, bu0, gu0, betau0,
           Wm1, bm1, gm1, betam1, Wu1, bu1, gu1, betau1,
           W1, b1, W2, b2):
    N = x.shape[0]
    E = edge_index.shape[1]
    src = edge_index[0]
    dst = edge_index[1]

    n_chunks = E // CHUNK
    n_pad = -(-n_chunks // NW) * NW

    def chunked(v):
        return jnp.pad(v.reshape(n_chunks, CHUNK),
                       ((0, n_pad - n_chunks), (0, 0)))

    src2d = chunked(src)
    dst2d = chunked(dst)

    def row(v):
        return v.reshape(1, -1)

    h, A, B = _node0(x, Wa, row(ba), Wm0[:D], Wm0[D:2 * D])
    Ee0, Ee1 = _edgefold(edge_attr, Wb, row(bb),
                         Wm0[2 * D:], row(bm0), Wm1[2 * D:], row(bm1))

    GA, GB = _sc_gather(A, B, src2d, dst2d, E)
    M = _msg(GA, GB, Ee0, row(gm0), row(betam0))
    P = _sc_scatter(M, dst, N)
    h, A, B = _update(P, h, Wu0[:D], Wu0[D:], row(bu0), row(gu0), row(betau0),
                      Wm1[:D], Wm1[D:2 * D])

    GA, GB = _sc_gather(A, B, src2d, dst2d, E)
    M = _msg(GA, GB, Ee1, row(gm1), row(betam1))
    P = _sc_scatter(M, dst, N)

    return _final(P, h, Wu1[:D], Wu1[D:], row(bu1), row(gu1), row(betau1),
                  W1, row(b1), W2.reshape(D, 1), b2.reshape(1, 1),
                  batch_idx.reshape(N, 1))


# speedup vs baseline: 1.7803x; 1.1368x over previous
import functools

import jax
import jax.numpy as jnp
from jax import lax
from jax.experimental import pallas as pl
from jax.experimental.pallas import tpu as pltpu
from jax.experimental.pallas import tpu_sc as plsc

D = 128
G = 256
EPS = 1e-5

NC = 2
NS = 16
NW = NC * NS
CHUNK = 128


def _ln_tanh(z, g, b):
    mu = jnp.mean(z, axis=-1, keepdims=True)
    var = jnp.mean((z - mu) ** 2, axis=-1, keepdims=True)
    return jnp.tanh((z - mu) * jax.lax.rsqrt(var + EPS) * g + b)


def _full(shape):
    return pl.BlockSpec(shape, lambda i: tuple(0 for _ in shape))


def _node0(x, Wa, ba, WmA0, WmB0):
    N = x.shape[0]
    R = 2000
    def body(x_r, Wa_r, ba_r, WmA_r, WmB_r, h_r, a_r, b_r):
        h = jnp.dot(x_r[...], Wa_r[...], preferred_element_type=jnp.float32) + ba_r[...]
        h_r[...] = h
        a_r[...] = jnp.dot(h, WmA_r[...], preferred_element_type=jnp.float32)
        b_r[...] = jnp.dot(h, WmB_r[...], preferred_element_type=jnp.float32)
    out = jax.ShapeDtypeStruct((N, D), jnp.float32)
    return pl.pallas_call(
        body,
        grid=(N // R,),
        in_specs=[pl.BlockSpec((R, D), lambda i: (i, 0)), _full((D, D)),
                  _full((1, D)), _full((D, D)), _full((D, D))],
        out_specs=[pl.BlockSpec((R, D), lambda i: (i, 0))] * 3,
        out_shape=[out, out, out],
    )(x, Wa, ba, WmA0, WmB0)


def _edgefold(ea, Wb, bb, WmC0, bm0, WmC1, bm1):
    E, DE = ea.shape
    R = 4000
    def body(ea_r, Wb_r, bb_r, C0_r, b0_r, C1_r, b1_r, e0_r, e1_r):
        e = jnp.dot(ea_r[...], Wb_r[...], preferred_element_type=jnp.float32) + bb_r[...]
        e0_r[...] = jnp.dot(e, C0_r[...], preferred_element_type=jnp.float32) + b0_r[...]
        e1_r[...] = jnp.dot(e, C1_r[...], preferred_element_type=jnp.float32) + b1_r[...]
    out = jax.ShapeDtypeStruct((E, D), jnp.float32)
    return pl.pallas_call(
        body,
        grid=(E // R,),
        in_specs=[pl.BlockSpec((R, DE), lambda i: (i, 0)), _full((DE, DE)),
                  _full((1, DE)), _full((DE, D)), _full((1, D)),
                  _full((DE, D)), _full((1, D))],
        out_specs=[pl.BlockSpec((R, D), lambda i: (i, 0))] * 2,
        out_shape=[out, out],
    )(ea, Wb, bb, WmC0, bm0, WmC1, bm1)


def _msg(s, ee, gm, bem):
    E = s.shape[0]
    R = 4000
    def body(s_r, ee_r, g_r, b_r, m_r):
        z = s_r[...] + ee_r[...]
        m_r[...] = _ln_tanh(z, g_r[...], b_r[...])
    return pl.pallas_call(
        body,
        grid=(E // R,),
        in_specs=[pl.BlockSpec((R, D), lambda i: (i, 0))] * 2 + [_full((1, D))] * 2,
        out_specs=pl.BlockSpec((R, D), lambda i: (i, 0)),
        out_shape=jax.ShapeDtypeStruct((E, D), jnp.float32),
    )(s, ee, gm, bem)


def _update(P, h, WuA, WuB, bu, gu, beu, WmA, WmB):
    N = h.shape[0]
    R = 2000
    def body(P_r, h_r, WuA_r, WuB_r, bu_r, gu_r, beu_r, WmA_r, WmB_r,
             h1_r, a_r, b_r):
        aggr = P_r[0] + P_r[1]
        z = (jnp.dot(aggr, WuA_r[...], preferred_element_type=jnp.float32)
             + jnp.dot(h_r[...], WuB_r[...], preferred_element_type=jnp.float32)
             + bu_r[...])
        h1 = _ln_tanh(z, gu_r[...], beu_r[...])
        h1_r[...] = h1
        a_r[...] = jnp.dot(h1, WmA_r[...], preferred_element_type=jnp.float32)
        b_r[...] = jnp.dot(h1, WmB_r[...], preferred_element_type=jnp.float32)
    out = jax.ShapeDtypeStruct((N, D), jnp.float32)
    return pl.pallas_call(
        body,
        grid=(N // R,),
        in_specs=[pl.BlockSpec((2, R, D), lambda i: (0, i, 0)),
                  pl.BlockSpec((R, D), lambda i: (i, 0)),
                  _full((D, D)), _full((D, D)), _full((1, D)),
                  _full((1, D)), _full((1, D)), _full((D, D)), _full((D, D))],
        out_specs=[pl.BlockSpec((R, D), lambda i: (i, 0))] * 3,
        out_shape=[out, out, out],
    )(P, h, WuA, WuB, bu, gu, beu, WmA, WmB)


def _final(P, h, WuA, WuB, bu, gu, beu, W1, b1, W2, b2, batch2d):
    N = h.shape[0]
    R = 2000
    nblk = N // R
    def body(P_r, h_r, WuA_r, WuB_r, bu_r, gu_r, beu_r,
             W1_r, b1_r, W2_r, b2_r, bi_r, out_r, sums, cnts):
        i = pl.program_id(0)
        aggr = P_r[0] + P_r[1]
        z = (jnp.dot(aggr, WuA_r[...], preferred_element_type=jnp.float32)
             + jnp.dot(h_r[...], WuB_r[...], preferred_element_type=jnp.float32)
             + bu_r[...])
        h2 = _ln_tanh(z, gu_r[...], beu_r[...])
        hid = jax.nn.relu(jnp.dot(h2, W1_r[...], preferred_element_type=jnp.float32)
                          + b1_r[...])
        r = jnp.dot(hid, W2_r[...], preferred_element_type=jnp.float32) + b2_r[...]
        gids = jax.lax.broadcasted_iota(jnp.int32, (R, G), 1)
        mask = (bi_r[...] == gids).astype(jnp.float32)
        blk_sum = jax.lax.dot_general(
            r, mask, (((0,), (0,)), ((), ())), preferred_element_type=jnp.float32)
        blk_cnt = jnp.sum(mask, axis=0, keepdims=True)

        @pl.when(i == 0)
        def _():
            sums[...] = jnp.zeros_like(sums)
            cnts[...] = jnp.zeros_like(cnts)
        sums[...] += blk_sum
        cnts[...] += blk_cnt

        @pl.when(i == nblk - 1)
        def _():
            out_r[...] = sums[...] / jnp.maximum(cnts[...], 1.0)
    return pl.pallas_call(
        body,
        grid=(nblk,),
        in_specs=[pl.BlockSpec((2, R, D), lambda i: (0, i, 0)),
                  pl.BlockSpec((R, D), lambda i: (i, 0)),
                  _full((D, D)), _full((D, D)), _full((1, D)),
                  _full((1, D)), _full((1, D)),
                  _full((D, D)), _full((1, D)), _full((D, 1)), _full((1, 1)),
                  pl.BlockSpec((R, 1), lambda i: (i, 0))],
        out_specs=_full((1, G)),
        out_shape=jax.ShapeDtypeStruct((1, G), jnp.float32),
        scratch_shapes=[pltpu.VMEM((1, G), jnp.float32),
                        pltpu.VMEM((1, G), jnp.float32)],
    )(P, h, WuA, WuB, bu, gu, beu, W1, b1, W2, b2, batch2d)


def _sc_gather(A, B, src2d, dst2d, E):
    n_pad, C = src2d.shape
    PER = n_pad // NW
    n_chunks = E // C
    NB = 3
    mesh = plsc.VectorSubcoreMesh(core_axis_name="c", subcore_axis_name="s",
                                  num_cores=NC, num_subcores=NS)
    out = jax.ShapeDtypeStruct((E, D), jnp.float32)

    @functools.partial(
        pl.kernel, out_type=out, mesh=mesh,
        scratch_types=[
            pltpu.VMEM((PER, C), jnp.int32),
            pltpu.VMEM((PER, C), jnp.int32),
            pltpu.VMEM((NB, C, D), jnp.float32),
            pltpu.VMEM((NB, C, D), jnp.float32),
            [pltpu.SemaphoreType.DMA] * NB,
            [pltpu.SemaphoreType.DMA] * NB,
            [pltpu.SemaphoreType.DMA] * NB,
        ])
    def k(A_h, B_h, src_h, dst_h, S_h,
          idxs, idxd, bufa, bufb, sga, sgb, swa):
        c = lax.axis_index("c")
        s = lax.axis_index("s")
        wid = s * NC + c
        chunk0 = wid * PER
        nvalid = jnp.minimum(jnp.maximum(n_chunks - chunk0, 0), PER)

        pltpu.sync_copy(src_h.at[pl.ds(chunk0, PER)], idxs)
        pltpu.sync_copy(dst_h.at[pl.ds(chunk0, PER)], idxd)

        def fire_gather(kk, b):
            pltpu.async_copy(A_h.at[idxs.at[kk]], bufa.at[b], sga[b])
            pltpu.async_copy(B_h.at[idxd.at[kk]], bufb.at[b], sgb[b])

        def drain_gather(kk, b):
            pltpu.make_async_copy(A_h.at[idxs.at[kk]], bufa.at[b], sga[b]).wait()
            pltpu.make_async_copy(B_h.at[idxd.at[kk]], bufb.at[b], sgb[b]).wait()

        def fire_wb(kk, b):
            base = (chunk0 + kk) * C
            pltpu.async_copy(bufa.at[b], S_h.at[pl.ds(base, C)], swa[b])

        def drain_wb(kk, b):
            base = (chunk0 + kk) * C
            pltpu.make_async_copy(bufa.at[b], S_h.at[pl.ds(base, C)], swa[b]).wait()

        def add_chunk(b):
            def arow(r, carry):
                for j in range(D // 16):
                    sl = pl.ds(j * 16, 16)
                    bufa[b, r, sl] = bufa[b, r, sl] + bufb[b, r, sl]
                return carry

            lax.fori_loop(0, C, arow, 0)

        @pl.when(nvalid > 0)
        def _():
            fire_gather(0, 0)

        @pl.when(nvalid > 1)
        def _():
            fire_gather(1, 1)

        def outer(i, carry):
            for bb in range(NB):
                kk = i * NB + bb
                bn = (bb + 2) % NB

                @pl.when(kk < nvalid)
                def _():
                    drain_gather(kk, bb)
                    add_chunk(bb)
                    fire_wb(kk, bb)

                    @pl.when((kk + 2 < nvalid) & (kk > 0))
                    def _():
                        drain_wb(kk - 1, bn)
                        fire_gather(kk + 2, bn)

                    @pl.when((kk + 2 < nvalid) & (kk == 0))
                    def _():
                        fire_gather(kk + 2, bn)
            return carry

        lax.fori_loop(0, -(-(PER + 2) // NB), outer, 0)

        for bb in range(NB):
            ct = nvalid - 1 - lax.rem(nvalid - 1 - bb + 2 * NB, NB)

            @pl.when((ct >= 0) & (ct >= nvalid - NB))
            def _():
                drain_wb(ct, bb)

    return k(A, B, src2d, dst2d)


def _sc_scatter(M, dst, N):
    E = dst.shape[0]
    C = CHUNK
    n_chunks = E // C
    PER = -(-n_chunks // NW)
    row_step = 624
    row_span = 640
    mesh = plsc.VectorSubcoreMesh(core_axis_name="c", subcore_axis_name="s",
                                  num_cores=NC, num_subcores=NS)

    @functools.partial(
        pl.kernel, out_type=jax.ShapeDtypeStruct((NC, N, D), jnp.float32),
        mesh=mesh,
        scratch_types=[
            pltpu.VMEM_SHARED((N, D), jnp.float32),
            pltpu.VMEM((2, C, D), jnp.float32),
            pltpu.VMEM((C,), jnp.int32),
            pltpu.VMEM((C,), jnp.int32),
            [pltpu.SemaphoreType.DMA] * 2,
            [pltpu.SemaphoreType.DMA] * 2,
        ])
    def k(M_h, dst_h, P_h, aggr, bufm, idx0, idx1, sm, si):
        c = lax.axis_index("c")
        s = lax.axis_index("s")
        wid = s * NC + c
        chunk0 = wid * PER
        nvalid = jnp.minimum(jnp.maximum(n_chunks - chunk0, 0), PER)
        idxb = (idx0, idx1)

        def zero_row(i, carry):
            for j in range(D // 16):
                bufm[0, i, pl.ds(j * 16, 16)] = jnp.zeros((16,), jnp.float32)
            return carry

        lax.fori_loop(0, C, zero_row, 0)
        for kk in range(row_span // C):
            pltpu.sync_copy(bufm.at[0],
                            aggr.at[pl.ds(s * row_step + kk * C, C)])
        plsc.subcore_barrier()

        def fire(kk, b):
            base = (chunk0 + kk) * C
            pltpu.async_copy(M_h.at[pl.ds(base, C)], bufm.at[b], sm[b])
            pltpu.async_copy(dst_h.at[pl.ds(base, C)], idxb[b], si[b])

        def drain(kk, b):
            base = (chunk0 + kk) * C
            pltpu.make_async_copy(M_h.at[pl.ds(base, C)], bufm.at[b], sm[b]).wait()
            pltpu.make_async_copy(dst_h.at[pl.ds(base, C)], idxb[b], si[b]).wait()

        @pl.when(nvalid > 0)
        def _():
            fire(0, 0)

        @pl.when(nvalid > 1)
        def _():
            fire(1, 1)

        def outer(i, carry):
            for b in range(2):
                kk = i * 2 + b

                @pl.when(kk < nvalid)
                def _():
                    drain(kk, b)
                    pltpu.sync_copy(bufm.at[b], aggr.at[idxb[b]], add=True)

                    @pl.when(kk + 2 < nvalid)
                    def _():
                        fire(kk + 2, b)
            return carry

        lax.fori_loop(0, -(-PER // 2), outer, 0)
        plsc.subcore_barrier()
        pltpu.sync_copy(aggr.at[pl.ds(s * row_step, row_span)],
                        P_h.at[c, pl.ds(s * row_step, row_span)])

    return k(M, dst)


def kernel(x, edge_index, edge_attr, batch_idx, Wa, ba, Wb, bb,
           Wm0, bm0, gm0, betam0, Wu0, bu0, gu0, betau0,
           Wm1, bm1, gm1, betam1, Wu1, bu1, gu1, betau1,
           W1, b1, W2, b2):
    N = x.shape[0]
    E = edge_index.shape[1]
    src = edge_index[0]
    dst = edge_index[1]

    n_chunks = E // CHUNK
    n_pad = -(-n_chunks // NW) * NW

    def chunked(v):
        return jnp.pad(v.reshape(n_chunks, CHUNK),
                       ((0, n_pad - n_chunks), (0, 0)))

    src2d = chunked(src)
    dst2d = chunked(dst)

    def row(v):
        return v.reshape(1, -1)

    h, A, B = _node0(x, Wa, row(ba), Wm0[:D], Wm0[D:2 * D])
    Ee0, Ee1 = _edgefold(edge_attr, Wb, row(bb),
                         Wm0[2 * D:], row(bm0), Wm1[2 * D:], row(bm1))

    S = _sc_gather(A, B, src2d, dst2d, E)
    M = _msg(S, Ee0, row(gm0), row(betam0))
    P = _sc_scatter(M, dst, N)
    h, A, B = _update(P, h, Wu0[:D], Wu0[D:], row(bu0), row(gu0), row(betau0),
                      Wm1[:D], Wm1[D:2 * D])

    S = _sc_gather(A, B, src2d, dst2d, E)
    M = _msg(S, Ee1, row(gm1), row(betam1))
    P = _sc_scatter(M, dst, N)

    return _final(P, h, Wu1[:D], Wu1[D:], row(bu1), row(gu1), row(betau1),
                  W1, row(b1), W2.reshape(D, 1), b2.reshape(1, 1),
                  batch_idx.reshape(N, 1))


# speedup vs baseline: 1.9953x; 1.1208x over previous
import functools

import jax
import jax.numpy as jnp
from jax import lax
from jax.experimental import pallas as pl
from jax.experimental.pallas import tpu as pltpu
from jax.experimental.pallas import tpu_sc as plsc

D = 128
G = 256
EPS = 1e-5

NC = 2
NS = 16
NW = NC * NS
CHUNK = 128


def _ln_tanh(z, g, b):
    mu = jnp.mean(z, axis=-1, keepdims=True)
    var = jnp.mean((z - mu) ** 2, axis=-1, keepdims=True)
    return jnp.tanh((z - mu) * jax.lax.rsqrt(var + EPS) * g + b)


def _full(shape):
    return pl.BlockSpec(shape, lambda i: tuple(0 for _ in shape))


def _node0(x, Wa, ba, WmA0, WmB0):
    N = x.shape[0]
    R = 2000
    def body(x_r, Wa_r, ba_r, WmA_r, WmB_r, h_r, a_r, b_r):
        h = jnp.dot(x_r[...], Wa_r[...], preferred_element_type=jnp.float32) + ba_r[...]
        h_r[...] = h
        a_r[...] = jnp.dot(h, WmA_r[...], preferred_element_type=jnp.float32)
        b_r[...] = jnp.dot(h, WmB_r[...], preferred_element_type=jnp.float32)
    out = jax.ShapeDtypeStruct((N, D), jnp.float32)
    return pl.pallas_call(
        body,
        grid=(N // R,),
        in_specs=[pl.BlockSpec((R, D), lambda i: (i, 0)), _full((D, D)),
                  _full((1, D)), _full((D, D)), _full((D, D))],
        out_specs=[pl.BlockSpec((R, D), lambda i: (i, 0))] * 3,
        out_shape=[out, out, out],
    )(x, Wa, ba, WmA0, WmB0)


def _msg(s, ea, Wb, bb, WmC, bm, gm, bem):
    E, DE = ea.shape
    R = 4000
    def body(s_r, ea_r, Wb_r, bb_r, C_r, bm_r, g_r, b_r, m_r):
        e = jnp.dot(ea_r[...], Wb_r[...], preferred_element_type=jnp.float32) + bb_r[...]
        ee = jnp.dot(e, C_r[...], preferred_element_type=jnp.float32) + bm_r[...]
        z = s_r[...] + ee
        m_r[...] = _ln_tanh(z, g_r[...], b_r[...])
    return pl.pallas_call(
        body,
        grid=(E // R,),
        in_specs=[pl.BlockSpec((R, D), lambda i: (i, 0)),
                  pl.BlockSpec((R, DE), lambda i: (i, 0)),
                  _full((DE, DE)), _full((1, DE)), _full((DE, D)),
                  _full((1, D)), _full((1, D)), _full((1, D))],
        out_specs=pl.BlockSpec((R, D), lambda i: (i, 0)),
        out_shape=jax.ShapeDtypeStruct((E, D), jnp.float32),
    )(s, ea, Wb, bb, WmC, bm, gm, bem)


def _update(P, h, WuA, WuB, bu, gu, beu, WmA, WmB):
    N = h.shape[0]
    R = 2000
    def body(P_r, h_r, WuA_r, WuB_r, bu_r, gu_r, beu_r, WmA_r, WmB_r,
             h1_r, a_r, b_r):
        aggr = P_r[0] + P_r[1]
        z = (jnp.dot(aggr, WuA_r[...], preferred_element_type=jnp.float32)
             + jnp.dot(h_r[...], WuB_r[...], preferred_element_type=jnp.float32)
             + bu_r[...])
        h1 = _ln_tanh(z, gu_r[...], beu_r[...])
        h1_r[...] = h1
        a_r[...] = jnp.dot(h1, WmA_r[...], preferred_element_type=jnp.float32)
        b_r[...] = jnp.dot(h1, WmB_r[...], preferred_element_type=jnp.float32)
    out = jax.ShapeDtypeStruct((N, D), jnp.float32)
    return pl.pallas_call(
        body,
        grid=(N // R,),
        in_specs=[pl.BlockSpec((2, R, D), lambda i: (0, i, 0)),
                  pl.BlockSpec((R, D), lambda i: (i, 0)),
                  _full((D, D)), _full((D, D)), _full((1, D)),
                  _full((1, D)), _full((1, D)), _full((D, D)), _full((D, D))],
        out_specs=[pl.BlockSpec((R, D), lambda i: (i, 0))] * 3,
        out_shape=[out, out, out],
    )(P, h, WuA, WuB, bu, gu, beu, WmA, WmB)


def _final(P, h, WuA, WuB, bu, gu, beu, W1, b1, W2, b2, batch2d):
    N = h.shape[0]
    R = 2000
    nblk = N // R
    def body(P_r, h_r, WuA_r, WuB_r, bu_r, gu_r, beu_r,
             W1_r, b1_r, W2_r, b2_r, bi_r, out_r, sums, cnts):
        i = pl.program_id(0)
        aggr = P_r[0] + P_r[1]
        z = (jnp.dot(aggr, WuA_r[...], preferred_element_type=jnp.float32)
             + jnp.dot(h_r[...], WuB_r[...], preferred_element_type=jnp.float32)
             + bu_r[...])
        h2 = _ln_tanh(z, gu_r[...], beu_r[...])
        hid = jax.nn.relu(jnp.dot(h2, W1_r[...], preferred_element_type=jnp.float32)
                          + b1_r[...])
        r = jnp.dot(hid, W2_r[...], preferred_element_type=jnp.float32) + b2_r[...]
        gids = jax.lax.broadcasted_iota(jnp.int32, (R, G), 1)
        mask = (bi_r[...] == gids).astype(jnp.float32)
        blk_sum = jax.lax.dot_general(
            r, mask, (((0,), (0,)), ((), ())), preferred_element_type=jnp.float32)
        blk_cnt = jnp.sum(mask, axis=0, keepdims=True)

        @pl.when(i == 0)
        def _():
            sums[...] = jnp.zeros_like(sums)
            cnts[...] = jnp.zeros_like(cnts)
        sums[...] += blk_sum
        cnts[...] += blk_cnt

        @pl.when(i == nblk - 1)
        def _():
            out_r[...] = sums[...] / jnp.maximum(cnts[...], 1.0)
    return pl.pallas_call(
        body,
        grid=(nblk,),
        in_specs=[pl.BlockSpec((2, R, D), lambda i: (0, i, 0)),
                  pl.BlockSpec((R, D), lambda i: (i, 0)),
                  _full((D, D)), _full((D, D)), _full((1, D)),
                  _full((1, D)), _full((1, D)),
                  _full((D, D)), _full((1, D)), _full((D, 1)), _full((1, 1)),
                  pl.BlockSpec((R, 1), lambda i: (i, 0))],
        out_specs=_full((1, G)),
        out_shape=jax.ShapeDtypeStruct((1, G), jnp.float32),
        scratch_shapes=[pltpu.VMEM((1, G), jnp.float32),
                        pltpu.VMEM((1, G), jnp.float32)],
    )(P, h, WuA, WuB, bu, gu, beu, W1, b1, W2, b2, batch2d)


def _sc_gather(A, B, src2d, dst2d, E):
    n_pad, C = src2d.shape
    PER = n_pad // NW
    n_chunks = E // C
    NB = 3
    mesh = plsc.VectorSubcoreMesh(core_axis_name="c", subcore_axis_name="s",
                                  num_cores=NC, num_subcores=NS)
    out = jax.ShapeDtypeStruct((E, D), jnp.float32)

    @functools.partial(
        pl.kernel, out_type=out, mesh=mesh,
        scratch_types=[
            pltpu.VMEM((PER, C), jnp.int32),
            pltpu.VMEM((PER, C), jnp.int32),
            pltpu.VMEM((NB, C, D), jnp.float32),
            pltpu.VMEM((NB, C, D), jnp.float32),
            [pltpu.SemaphoreType.DMA] * NB,
            [pltpu.SemaphoreType.DMA] * NB,
            [pltpu.SemaphoreType.DMA] * NB,
        ])
    def k(A_h, B_h, src_h, dst_h, S_h,
          idxs, idxd, bufa, bufb, sga, sgb, swa):
        c = lax.axis_index("c")
        s = lax.axis_index("s")
        wid = s * NC + c
        chunk0 = wid * PER
        nvalid = jnp.minimum(jnp.maximum(n_chunks - chunk0, 0), PER)

        pltpu.sync_copy(src_h.at[pl.ds(chunk0, PER)], idxs)
        pltpu.sync_copy(dst_h.at[pl.ds(chunk0, PER)], idxd)

        def fire_gather(kk, b):
            pltpu.async_copy(A_h.at[idxs.at[kk]], bufa.at[b], sga[b])
            pltpu.async_copy(B_h.at[idxd.at[kk]], bufb.at[b], sgb[b])

        def drain_gather(kk, b):
            pltpu.make_async_copy(A_h.at[idxs.at[kk]], bufa.at[b], sga[b]).wait()
            pltpu.make_async_copy(B_h.at[idxd.at[kk]], bufb.at[b], sgb[b]).wait()

        def fire_wb(kk, b):
            base = (chunk0 + kk) * C
            pltpu.async_copy(bufa.at[b], S_h.at[pl.ds(base, C)], swa[b])

        def drain_wb(kk, b):
            base = (chunk0 + kk) * C
            pltpu.make_async_copy(bufa.at[b], S_h.at[pl.ds(base, C)], swa[b]).wait()

        def add_chunk(b):
            def arow(r, carry):
                for j in range(D // 16):
                    sl = pl.ds(j * 16, 16)
                    bufa[b, r, sl] = bufa[b, r, sl] + bufb[b, r, sl]
                return carry

            lax.fori_loop(0, C, arow, 0)

        @pl.when(nvalid > 0)
        def _():
            fire_gather(0, 0)

        @pl.when(nvalid > 1)
        def _():
            fire_gather(1, 1)

        def outer(i, carry):
            for bb in range(NB):
                kk = i * NB + bb
                bn = (bb + 2) % NB

                @pl.when(kk < nvalid)
                def _():
                    drain_gather(kk, bb)
                    add_chunk(bb)
                    fire_wb(kk, bb)

                    @pl.when((kk + 2 < nvalid) & (kk > 0))
                    def _():
                        drain_wb(kk - 1, bn)
                        fire_gather(kk + 2, bn)

                    @pl.when((kk + 2 < nvalid) & (kk == 0))
                    def _():
                        fire_gather(kk + 2, bn)
            return carry

        lax.fori_loop(0, -(-(PER + 2) // NB), outer, 0)

        for bb in range(NB):
            ct = nvalid - 1 - lax.rem(nvalid - 1 - bb + 2 * NB, NB)

            @pl.when((ct >= 0) & (ct >= nvalid - NB))
            def _():
                drain_wb(ct, bb)

    return k(A, B, src2d, dst2d)


def _sc_scatter(M, dst, N):
    E = dst.shape[0]
    C = CHUNK
    n_chunks = E // C
    PER = -(-n_chunks // NW)
    row_step = 624
    row_span = 640
    mesh = plsc.VectorSubcoreMesh(core_axis_name="c", subcore_axis_name="s",
                                  num_cores=NC, num_subcores=NS)

    @functools.partial(
        pl.kernel, out_type=jax.ShapeDtypeStruct((NC, N, D), jnp.float32),
        mesh=mesh,
        scratch_types=[
            pltpu.VMEM_SHARED((N, D), jnp.float32),
            pltpu.VMEM((2, C, D), jnp.float32),
            pltpu.VMEM((C,), jnp.int32),
            pltpu.VMEM((C,), jnp.int32),
            [pltpu.SemaphoreType.DMA] * 2,
            [pltpu.SemaphoreType.DMA] * 2,
        ])
    def k(M_h, dst_h, P_h, aggr, bufm, idx0, idx1, sm, si):
        c = lax.axis_index("c")
        s = lax.axis_index("s")
        wid = s * NC + c
        chunk0 = wid * PER
        nvalid = jnp.minimum(jnp.maximum(n_chunks - chunk0, 0), PER)
        idxb = (idx0, idx1)

        def zero_row(i, carry):
            for j in range(D // 16):
                bufm[0, i, pl.ds(j * 16, 16)] = jnp.zeros((16,), jnp.float32)
            return carry

        lax.fori_loop(0, C, zero_row, 0)
        for kk in range(row_span // C):
            pltpu.sync_copy(bufm.at[0],
                            aggr.at[pl.ds(s * row_step + kk * C, C)])
        plsc.subcore_barrier()

        def fire(kk, b):
            base = (chunk0 + kk) * C
            pltpu.async_copy(M_h.at[pl.ds(base, C)], bufm.at[b], sm[b])
            pltpu.async_copy(dst_h.at[pl.ds(base, C)], idxb[b], si[b])

        def drain(kk, b):
            base = (chunk0 + kk) * C
            pltpu.make_async_copy(M_h.at[pl.ds(base, C)], bufm.at[b], sm[b]).wait()
            pltpu.make_async_copy(dst_h.at[pl.ds(base, C)], idxb[b], si[b]).wait()

        @pl.when(nvalid > 0)
        def _():
            fire(0, 0)

        @pl.when(nvalid > 1)
        def _():
            fire(1, 1)

        def outer(i, carry):
            for b in range(2):
                kk = i * 2 + b

                @pl.when(kk < nvalid)
                def _():
                    drain(kk, b)
                    pltpu.sync_copy(bufm.at[b], aggr.at[idxb[b]], add=True)

                    @pl.when(kk + 2 < nvalid)
                    def _():
                        fire(kk + 2, b)
            return carry

        lax.fori_loop(0, -(-PER // 2), outer, 0)
        plsc.subcore_barrier()
        pltpu.sync_copy(aggr.at[pl.ds(s * row_step, row_span)],
                        P_h.at[c, pl.ds(s * row_step, row_span)])

    return k(M, dst)


def kernel(x, edge_index, edge_attr, batch_idx, Wa, ba, Wb, bb,
           Wm0, bm0, gm0, betam0, Wu0, bu0, gu0, betau0,
           Wm1, bm1, gm1, betam1, Wu1, bu1, gu1, betau1,
           W1, b1, W2, b2):
    N = x.shape[0]
    E = edge_index.shape[1]
    src = edge_index[0]
    dst = edge_index[1]

    n_chunks = E // CHUNK
    n_pad = -(-n_chunks // NW) * NW

    def chunked(v):
        return jnp.pad(v.reshape(n_chunks, CHUNK),
                       ((0, n_pad - n_chunks), (0, 0)))

    src2d = chunked(src)
    dst2d = chunked(dst)

    def row(v):
        return v.reshape(1, -1)

    h, A, B = _node0(x, Wa, row(ba), Wm0[:D], Wm0[D:2 * D])

    S = _sc_gather(A, B, src2d, dst2d, E)
    M = _msg(S, edge_attr, Wb, row(bb), Wm0[2 * D:], row(bm0),
             row(gm0), row(betam0))
    P = _sc_scatter(M, dst, N)
    h, A, B = _update(P, h, Wu0[:D], Wu0[D:], row(bu0), row(gu0), row(betau0),
                      Wm1[:D], Wm1[D:2 * D])

    S = _sc_gather(A, B, src2d, dst2d, E)
    M = _msg(S, edge_attr, Wb, row(bb), Wm1[2 * D:], row(bm1),
             row(gm1), row(betam1))
    P = _sc_scatter(M, dst, N)

    return _final(P, h, Wu1[:D], Wu1[D:], row(bu1), row(gu1), row(betau1),
                  W1, row(b1), W2.reshape(D, 1), b2.reshape(1, 1),
                  batch_idx.reshape(N, 1))
